# Initial kernel scaffold; baseline (speedup 1.0000x reference)
import jax
import jax.numpy as jnp
from jax.experimental import pallas as pl


def kernel(x, edge_index, edge_dist, edge_dist_count, batch_idx, W_in1, b_in1, W_in2, b_in2, Wq, bq, Wk, bk, Wv, bv, W_skip, b_skip, spatial_emb, Wf1, bf1, Wf2, bf2, Wo, bo):
    raise NotImplementedError("write your pallas kernel here")


# speedup vs baseline: 7.0671x; 7.0671x over previous
import functools
import math

import jax
import jax.numpy as jnp
from jax import lax
from jax.experimental import pallas as pl
from jax.experimental.pallas import tpu as pltpu
from jax.experimental.pallas import tpu_sc as plsc

_N = 10000
_E = 320000
_D = 128
_NC = 2
_NS = 16
_NW = _NC * _NS
_EPT = _E // _NW
_B = 80
_NCHUNK = _EPT // _B
_GRP = _B // 16
_NP = 10240
_ZR = 80
_RPT = _NP // _NS
_BLK = 1000
_SCALE = 1.0 / math.sqrt(128.0)

_dot = functools.partial(jnp.dot, precision=lax.Precision.HIGHEST,
                         preferred_element_type=jnp.float32)


def _mlp_qkv_body(x_ref, w1_ref, b1_ref, w2_ref, b2_ref, wq_ref, bq_ref,
                  wk_ref, bk_ref, wv_ref, bv_ref,
                  h_ref, q_ref, k_ref, v_ref):
    x = x_ref[...]
    h = jnp.maximum(_dot(x, w1_ref[...]) + b1_ref[...], 0.0)
    h = _dot(h, w2_ref[...]) + b2_ref[...]
    h_ref[...] = h
    q_ref[...] = (_dot(h, wq_ref[...]) + bq_ref[...]) * _SCALE
    k_ref[...] = _dot(h, wk_ref[...]) + bk_ref[...]
    v_ref[...] = _dot(h, wv_ref[...]) + bv_ref[...]


def _mlp_qkv(x, w1, b1, w2, b2, wq, bq, wk, bk, wv, bv):
    row = pl.BlockSpec((_BLK, _D), lambda i: (i, 0))
    wsp = pl.BlockSpec((_D, _D), lambda i: (0, 0))
    bsp = pl.BlockSpec((1, _D), lambda i: (0, 0))
    return pl.pallas_call(
        _mlp_qkv_body,
        grid=(_N // _BLK,),
        in_specs=[row, wsp, bsp, wsp, bsp, wsp, bsp, wsp, bsp, wsp, bsp],
        out_specs=[row, row, row, row],
        out_shape=[jax.ShapeDtypeStruct((_N, _D), jnp.float32)] * 4,
    )(x, w1, b1, w2, b2, wq, bq, wk, bk, wv, bv)


def _sc_body(src_hbm, dst_hbm, eb_hbm, q_hbm, k_hbm, v_hbm,
             z128_hbm, numer_hbm, denom_hbm,
             src_v, dst_v, dst2_v, eb_v, a_v, k_v, w_v,
             acc_n, acc_d, sem_q, sem_k, sem_v):
    cid = lax.axis_index("c")
    sid = lax.axis_index("s")
    wid = cid * _NS + sid

    pltpu.sync_copy(z128_hbm, a_v)
    for j in range(_RPT // _ZR):
        pltpu.sync_copy(a_v, acc_n.at[pl.ds(sid * _RPT + j * _ZR, _ZR)])
    pltpu.sync_copy(a_v, acc_d.at[pl.ds(sid * _ZR, _ZR)])
    plsc.subcore_barrier()

    iota = lax.iota(jnp.int32, 16)
    e0 = wid * _EPT

    def chunk(i, carry):
        base = pl.multiple_of(e0 + i * _B, 8)
        pltpu.sync_copy(src_hbm.at[pl.ds(base, _B)], src_v)
        pltpu.sync_copy(dst_hbm.at[pl.ds(base, _B)], dst_v)
        pltpu.sync_copy(eb_hbm.at[pl.ds(base, _B)], eb_v)
        cq = pltpu.async_copy(q_hbm.at[dst_v], a_v, sem_q)
        ck = pltpu.async_copy(k_hbm.at[src_v], k_v, sem_k)
        cq.wait()
        ck.wait()

        def dot_group(g, carry):
            alpha16 = jnp.zeros((16,), jnp.float32)
            for l in range(16):
                e = g * 16 + l
                accs = [a_v[e, pl.ds(c * 16, 16)] * k_v[e, pl.ds(c * 16, 16)]
                        for c in range(4)]
                for c in range(4, 8):
                    accs[c % 4] = accs[c % 4] + (a_v[e, pl.ds(c * 16, 16)] *
                                                 k_v[e, pl.ds(c * 16, 16)])
                s = (accs[0] + accs[1]) + (accs[2] + accs[3])
                for sh in (8, 4, 2, 1):
                    s = s + jnp.take(s, jnp.bitwise_xor(iota, sh))
                alpha16 = jnp.where(iota == l, s, alpha16)
            sl16 = pl.ds(g * 16, 16)
            w_v[sl16] = jnp.exp(alpha16 + eb_v[sl16])
            return carry

        lax.fori_loop(0, _GRP, dot_group, 0)
        cv = pltpu.async_copy(v_hbm.at[src_v], a_v, sem_v)
        cv.wait()

        def scale_group(g, carry):
            sl16 = pl.ds(g * 16, 16)
            w16 = w_v[sl16]
            d16 = dst_v[sl16]
            dst2_v[sl16] = lax.shift_right_logical(d16, 3)
            for l in range(16):
                e = g * 16 + l
                we = w16[l]
                blk_e = jnp.bitwise_and(d16[l], 7)
                for c in range(8):
                    sl = pl.ds(c * 16, 16)
                    a_v[e, sl] = a_v[e, sl] * we
                    wsel = jnp.where(blk_e == c, we, 0.0)
                    k_v[e, sl] = jnp.where(iota == 0, wsel, 0.0)
            return carry

        lax.fori_loop(0, _GRP, scale_group, 0)
        pltpu.sync_copy(a_v, acc_n.at[dst_v], add=True)
        pltpu.sync_copy(k_v, acc_d.at[dst2_v], add=True)
        return carry

    lax.fori_loop(0, _NCHUNK, chunk, 0)

    plsc.subcore_barrier()
    for j in range(_RPT // _ZR):
        r0 = sid * _RPT + j * _ZR
        pltpu.sync_copy(acc_n.at[pl.ds(r0, _ZR)], a_v)
        pltpu.sync_copy(a_v, numer_hbm.at[cid, pl.ds(r0, _ZR)])
    pltpu.sync_copy(acc_d.at[pl.ds(sid * _ZR, _ZR)], k_v)
    pltpu.sync_copy(k_v, denom_hbm.at[cid, pl.ds(sid * _ZR, _ZR)])


_sc_edge = functools.partial(
    pl.kernel,
    out_type=[
        jax.ShapeDtypeStruct((_NC, _NP, _D), jnp.float32),
        jax.ShapeDtypeStruct((_NC, _NP // 8, _D), jnp.float32),
    ],
    mesh=plsc.VectorSubcoreMesh(core_axis_name="c", subcore_axis_name="s"),
    scratch_types=[
        pltpu.VMEM((_B,), jnp.int32),
        pltpu.VMEM((_B,), jnp.int32),
        pltpu.VMEM((_B,), jnp.int32),
        pltpu.VMEM((_B,), jnp.float32),
        pltpu.VMEM((_B, _D), jnp.float32),
        pltpu.VMEM((_B, _D), jnp.float32),
        pltpu.VMEM((_B,), jnp.float32),
        pltpu.VMEM_SHARED((_NP, _D), jnp.float32),
        pltpu.VMEM_SHARED((_NP // 8, _D), jnp.float32),
        pltpu.SemaphoreType.DMA,
        pltpu.SemaphoreType.DMA,
        pltpu.SemaphoreType.DMA,
    ],
)(_sc_body)


def _ebias_body(dist_ref, cnt_ref, se_ref, out_ref):
    dist = dist_ref[...]
    se = jnp.zeros(dist.shape, jnp.float32)
    for t in range(21):
        se = jnp.where(dist == t, se_ref[0, t], se)
    out_ref[...] = se - jnp.log(cnt_ref[...])


def _ebias(dist2d, cnt2d, se_row):
    return pl.pallas_call(
        _ebias_body,
        out_shape=jax.ShapeDtypeStruct((_E // _D, _D), jnp.float32),
    )(dist2d, cnt2d, se_row)


def _combine_ff_body(n0_ref, n1_ref, d0_ref, d1_ref, h8_ref, wsk_ref, bsk_ref,
                     wf1_ref, bf1_ref, wf2_ref, bf2_ref, wo_ref, bo_ref,
                     out_ref):
    den = d0_ref[...] + d1_ref[...] + 1e-16
    o = (n0_ref[...] + n1_ref[...]) / den
    skip = _dot(h8_ref[...], wsk_ref[...]) + bsk_ref[...]
    o = o + skip[1:2, :]
    o = jnp.maximum(_dot(o, wf1_ref[...]) + bf1_ref[...], 0.0)
    o = jnp.maximum(_dot(o, wf2_ref[...]) + bf2_ref[...], 0.0)
    out_ref[...] = _dot(o, wo_ref[...]) + bo_ref[...]


def _combine_ff(n0, n1, d0, d1, h8, wsk, bsk, wf1, bf1, wf2, bf2, wo, bo):
    row = pl.BlockSpec((_BLK, _D), lambda i: (i, 0))
    den = pl.BlockSpec((_BLK, 1), lambda i: (i, 0))
    wsp = pl.BlockSpec((_D, _D), lambda i: (0, 0))
    bsp = pl.BlockSpec((1, _D), lambda i: (0, 0))
    h8sp = pl.BlockSpec((8, _D), lambda i: (0, 0))
    return pl.pallas_call(
        _combine_ff_body,
        grid=(_N // _BLK,),
        in_specs=[row, row, den, den, h8sp,
                  wsp, bsp, wsp, bsp, wsp, bsp, wsp, bsp],
        out_specs=row,
        out_shape=jax.ShapeDtypeStruct((_N, _D), jnp.float32),
    )(n0, n1, d0, d1, h8, wsk, bsk, wf1, bf1, wf2, bf2, wo, bo)


def kernel(x, edge_index, edge_dist, edge_dist_count, batch_idx,
           W_in1, b_in1, W_in2, b_in2, Wq, bq, Wk, bk, Wv, bv,
           W_skip, b_skip, spatial_emb, Wf1, bf1, Wf2, bf2, Wo, bo):
    h, q, k, v = _mlp_qkv(x, W_in1, b_in1.reshape(1, -1), W_in2,
                          b_in2.reshape(1, -1), Wq, bq.reshape(1, -1),
                          Wk, bk.reshape(1, -1), Wv, bv.reshape(1, -1))
    src = edge_index[0]
    dst = edge_index[1]
    se_row = jnp.zeros((1, _D), jnp.float32).at[0, : spatial_emb.shape[0]].set(
        spatial_emb[:, 0])
    ebias = _ebias(edge_dist.reshape(_E // _D, _D),
                   edge_dist_count.reshape(_E // _D, _D), se_row).reshape(_E)
    z128 = jnp.zeros((_ZR, _D), jnp.float32)
    numer, denomp = _sc_edge(src, dst, ebias, q, k, v, z128)
    numer = numer[:, :_N]
    denom = denomp.reshape(_NC, _NP // 8, 8, 16)[:, :, :, 0].reshape(
        _NC, _NP)[:, :_N, None]
    return _combine_ff(numer[0], numer[1], denom[0], denom[1], h[:8],
                       W_skip, b_skip.reshape(1, -1),
                       Wf1, bf1.reshape(1, -1), Wf2, bf2.reshape(1, -1),
                       Wo, bo.reshape(1, -1))


# speedup vs baseline: 8.2961x; 1.1739x over previous
import functools
import math

import jax
import jax.numpy as jnp
from jax import lax
from jax.experimental import pallas as pl
from jax.experimental.pallas import tpu as pltpu
from jax.experimental.pallas import tpu_sc as plsc

_N = 10000
_E = 320000
_D = 128
_NC = 2
_NS = 16
_NW = _NC * _NS
_EPT = _E // _NW
_B = 80
_NCHUNK = _EPT // _B
_GRP = _B // 16
_NP = 10240
_ZR = 80
_RPT = _NP // _NS
_BLK = 1000
_SCALE = 1.0 / math.sqrt(128.0)

_dot = functools.partial(jnp.dot, precision=lax.Precision.HIGHEST,
                         preferred_element_type=jnp.float32)


def _mlp_qkv_body(x_ref, w1_ref, b1_ref, w2_ref, b2_ref, wq_ref, bq_ref,
                  wk_ref, bk_ref, wv_ref, bv_ref,
                  h_ref, q_ref, k_ref, v_ref):
    x = x_ref[...]
    h = jnp.maximum(_dot(x, w1_ref[...]) + b1_ref[...], 0.0)
    h = _dot(h, w2_ref[...]) + b2_ref[...]
    h_ref[...] = h
    q_ref[...] = (_dot(h, wq_ref[...]) + bq_ref[...]) * _SCALE
    k_ref[...] = _dot(h, wk_ref[...]) + bk_ref[...]
    v_ref[...] = _dot(h, wv_ref[...]) + bv_ref[...]


def _mlp_qkv(x, w1, b1, w2, b2, wq, bq, wk, bk, wv, bv):
    row = pl.BlockSpec((_BLK, _D), lambda i: (i, 0))
    wsp = pl.BlockSpec((_D, _D), lambda i: (0, 0))
    bsp = pl.BlockSpec((1, _D), lambda i: (0, 0))
    return pl.pallas_call(
        _mlp_qkv_body,
        grid=(_N // _BLK,),
        in_specs=[row, wsp, bsp, wsp, bsp, wsp, bsp, wsp, bsp, wsp, bsp],
        out_specs=[row, row, row, row],
        out_shape=[jax.ShapeDtypeStruct((_N, _D), jnp.float32)] * 4,
    )(x, w1, b1, w2, b2, wq, bq, wk, bk, wv, bv)


def _sc_body(src_hbm, dst_hbm, eb_hbm, q_hbm, k_hbm, v_hbm,
             z128_hbm, numer_hbm, denom_hbm,
             src_v, dst_v, dst2_v, eb_v, a_v, k_v, v_v, w_v,
             acc_n, acc_d, sem_q, sem_k, sem_v, sem_m):
    cid = lax.axis_index("c")
    sid = lax.axis_index("s")
    wid = cid * _NS + sid

    pltpu.sync_copy(z128_hbm, a_v)
    for j in range(_RPT // _ZR):
        pltpu.sync_copy(a_v, acc_n.at[pl.ds(sid * _RPT + j * _ZR, _ZR)])
    pltpu.sync_copy(a_v, acc_d.at[pl.ds(sid * _ZR, _ZR)])
    plsc.subcore_barrier()

    iota = lax.iota(jnp.int32, 16)
    e0 = wid * _EPT

    def chunk(i, carry):
        base = pl.multiple_of(e0 + i * _B, 8)
        cs = pltpu.async_copy(src_hbm.at[pl.ds(base, _B)], src_v, sem_m)
        cd = pltpu.async_copy(dst_hbm.at[pl.ds(base, _B)], dst_v, sem_m)
        ce = pltpu.async_copy(eb_hbm.at[pl.ds(base, _B)], eb_v, sem_m)
        cs.wait()
        cd.wait()
        ce.wait()
        cq = pltpu.async_copy(q_hbm.at[dst_v], a_v, sem_q)
        ck = pltpu.async_copy(k_hbm.at[src_v], k_v, sem_k)
        cv = pltpu.async_copy(v_hbm.at[src_v], v_v, sem_v)
        cq.wait()
        ck.wait()

        def dot_group(g, carry):
            alpha16 = jnp.zeros((16,), jnp.float32)
            for l in range(16):
                e = g * 16 + l
                accs = [a_v[e, pl.ds(c * 16, 16)] * k_v[e, pl.ds(c * 16, 16)]
                        for c in range(4)]
                for c in range(4, 8):
                    accs[c % 4] = accs[c % 4] + (a_v[e, pl.ds(c * 16, 16)] *
                                                 k_v[e, pl.ds(c * 16, 16)])
                s = (accs[0] + accs[1]) + (accs[2] + accs[3])
                for sh in (8, 4, 2, 1):
                    s = s + jnp.take(s, jnp.bitwise_xor(iota, sh))
                alpha16 = jnp.where(iota == l, s, alpha16)
            sl16 = pl.ds(g * 16, 16)
            w_v[sl16] = jnp.exp(alpha16 + eb_v[sl16])
            return carry

        lax.fori_loop(0, _GRP, dot_group, 0)
        cv.wait()

        def scale_group(g, carry):
            sl16 = pl.ds(g * 16, 16)
            w16 = w_v[sl16]
            d16 = dst_v[sl16]
            dst2_v[sl16] = lax.shift_right_logical(d16, 3)
            for l in range(16):
                e = g * 16 + l
                we = w16[l]
                blk_e = jnp.bitwise_and(d16[l], 7)
                for c in range(8):
                    sl = pl.ds(c * 16, 16)
                    v_v[e, sl] = v_v[e, sl] * we
                    wsel = jnp.where(blk_e == c, we, 0.0)
                    k_v[e, sl] = jnp.where(iota == 0, wsel, 0.0)
            return carry

        lax.fori_loop(0, _GRP, scale_group, 0)
        pltpu.sync_copy(v_v, acc_n.at[dst_v], add=True)
        pltpu.sync_copy(k_v, acc_d.at[dst2_v], add=True)
        return carry

    lax.fori_loop(0, _NCHUNK, chunk, 0)

    plsc.subcore_barrier()
    for j in range(_RPT // _ZR):
        r0 = sid * _RPT + j * _ZR
        pltpu.sync_copy(acc_n.at[pl.ds(r0, _ZR)], a_v)
        pltpu.sync_copy(a_v, numer_hbm.at[cid, pl.ds(r0, _ZR)])
    pltpu.sync_copy(acc_d.at[pl.ds(sid * _ZR, _ZR)], k_v)
    pltpu.sync_copy(k_v, denom_hbm.at[cid, pl.ds(sid * _ZR, _ZR)])


_sc_edge = functools.partial(
    pl.kernel,
    out_type=[
        jax.ShapeDtypeStruct((_NC, _NP, _D), jnp.float32),
        jax.ShapeDtypeStruct((_NC, _NP // 8, _D), jnp.float32),
    ],
    mesh=plsc.VectorSubcoreMesh(core_axis_name="c", subcore_axis_name="s"),
    scratch_types=[
        pltpu.VMEM((_B,), jnp.int32),
        pltpu.VMEM((_B,), jnp.int32),
        pltpu.VMEM((_B,), jnp.int32),
        pltpu.VMEM((_B,), jnp.float32),
        pltpu.VMEM((_B, _D), jnp.float32),
        pltpu.VMEM((_B, _D), jnp.float32),
        pltpu.VMEM((_B, _D), jnp.float32),
        pltpu.VMEM((_B,), jnp.float32),
        pltpu.VMEM_SHARED((_NP, _D), jnp.float32),
        pltpu.VMEM_SHARED((_NP // 8, _D), jnp.float32),
        pltpu.SemaphoreType.DMA,
        pltpu.SemaphoreType.DMA,
        pltpu.SemaphoreType.DMA,
        pltpu.SemaphoreType.DMA,
    ],
)(_sc_body)


def _ebias_body(dist_ref, cnt_ref, se_ref, out_ref):
    dist = dist_ref[...]
    se = jnp.zeros(dist.shape, jnp.float32)
    for t in range(21):
        se = jnp.where(dist == t, se_ref[0, t], se)
    out_ref[...] = se - jnp.log(cnt_ref[...])


def _ebias(dist2d, cnt2d, se_row):
    return pl.pallas_call(
        _ebias_body,
        out_shape=jax.ShapeDtypeStruct((_E // _D, _D), jnp.float32),
    )(dist2d, cnt2d, se_row)


def _combine_ff_body(n0_ref, n1_ref, d0_ref, d1_ref, h8_ref, wsk_ref, bsk_ref,
                     wf1_ref, bf1_ref, wf2_ref, bf2_ref, wo_ref, bo_ref,
                     out_ref):
    den = d0_ref[...] + d1_ref[...] + 1e-16
    o = (n0_ref[...] + n1_ref[...]) / den
    skip = _dot(h8_ref[...], wsk_ref[...]) + bsk_ref[...]
    o = o + skip[1:2, :]
    o = jnp.maximum(_dot(o, wf1_ref[...]) + bf1_ref[...], 0.0)
    o = jnp.maximum(_dot(o, wf2_ref[...]) + bf2_ref[...], 0.0)
    out_ref[...] = _dot(o, wo_ref[...]) + bo_ref[...]


def _combine_ff(n0, n1, d0, d1, h8, wsk, bsk, wf1, bf1, wf2, bf2, wo, bo):
    row = pl.BlockSpec((_BLK, _D), lambda i: (i, 0))
    den = pl.BlockSpec((_BLK, 1), lambda i: (i, 0))
    wsp = pl.BlockSpec((_D, _D), lambda i: (0, 0))
    bsp = pl.BlockSpec((1, _D), lambda i: (0, 0))
    h8sp = pl.BlockSpec((8, _D), lambda i: (0, 0))
    return pl.pallas_call(
        _combine_ff_body,
        grid=(_N // _BLK,),
        in_specs=[row, row, den, den, h8sp,
                  wsp, bsp, wsp, bsp, wsp, bsp, wsp, bsp],
        out_specs=row,
        out_shape=jax.ShapeDtypeStruct((_N, _D), jnp.float32),
    )(n0, n1, d0, d1, h8, wsk, bsk, wf1, bf1, wf2, bf2, wo, bo)


def kernel(x, edge_index, edge_dist, edge_dist_count, batch_idx,
           W_in1, b_in1, W_in2, b_in2, Wq, bq, Wk, bk, Wv, bv,
           W_skip, b_skip, spatial_emb, Wf1, bf1, Wf2, bf2, Wo, bo):
    h, q, k, v = _mlp_qkv(x, W_in1, b_in1.reshape(1, -1), W_in2,
                          b_in2.reshape(1, -1), Wq, bq.reshape(1, -1),
                          Wk, bk.reshape(1, -1), Wv, bv.reshape(1, -1))
    src = edge_index[0]
    dst = edge_index[1]
    se_row = jnp.zeros((1, _D), jnp.float32).at[0, : spatial_emb.shape[0]].set(
        spatial_emb[:, 0])
    ebias = _ebias(edge_dist.reshape(_E // _D, _D),
                   edge_dist_count.reshape(_E // _D, _D), se_row).reshape(_E)
    z128 = jnp.zeros((_ZR, _D), jnp.float32)
    numer, denomp = _sc_edge(src, dst, ebias, q, k, v, z128)
    numer = numer[:, :_N]
    denom = denomp.reshape(_NC, _NP // 8, 8, 16)[:, :, :, 0].reshape(
        _NC, _NP)[:, :_N, None]
    return _combine_ff(numer[0], numer[1], denom[0], denom[1], h[:8],
                       W_skip, b_skip.reshape(1, -1),
                       Wf1, bf1.reshape(1, -1), Wf2, bf2.reshape(1, -1),
                       Wo, bo.reshape(1, -1))


# speedup vs baseline: 9.3051x; 1.1216x over previous
import functools
import math

import jax
import jax.numpy as jnp
from jax import lax
from jax.experimental import pallas as pl
from jax.experimental.pallas import tpu as pltpu
from jax.experimental.pallas import tpu_sc as plsc

_N = 10000
_E = 320000
_D = 128
_NC = 2
_NS = 16
_NW = _NC * _NS
_EPT = _E // _NW
_B = 80
_NCHUNK = _EPT // _B
_GRP = _B // 16
_NP = 10240
_ZR = 80
_RPT = _NP // _NS
_BLK = 1000
_SCALE = 1.0 / math.sqrt(128.0)

_dot = functools.partial(jnp.dot, precision=lax.Precision.HIGHEST,
                         preferred_element_type=jnp.float32)


def _mlp_qkv_body(x_ref, w1_ref, b1_ref, w2_ref, b2_ref, wq_ref, bq_ref,
                  wk_ref, bk_ref, wv_ref, bv_ref,
                  h_ref, q_ref, k_ref, v_ref):
    x = x_ref[...]
    h = jnp.maximum(_dot(x, w1_ref[...]) + b1_ref[...], 0.0)
    h = _dot(h, w2_ref[...]) + b2_ref[...]
    h_ref[...] = h
    q_ref[...] = (_dot(h, wq_ref[...]) + bq_ref[...]) * _SCALE
    k_ref[...] = _dot(h, wk_ref[...]) + bk_ref[...]
    v_ref[...] = _dot(h, wv_ref[...]) + bv_ref[...]


def _mlp_qkv(x, w1, b1, w2, b2, wq, bq, wk, bk, wv, bv):
    row = pl.BlockSpec((_BLK, _D), lambda i: (i, 0))
    wsp = pl.BlockSpec((_D, _D), lambda i: (0, 0))
    bsp = pl.BlockSpec((1, _D), lambda i: (0, 0))
    return pl.pallas_call(
        _mlp_qkv_body,
        grid=(_N // _BLK,),
        in_specs=[row, wsp, bsp, wsp, bsp, wsp, bsp, wsp, bsp, wsp, bsp],
        out_specs=[row, row, row, row],
        out_shape=[jax.ShapeDtypeStruct((_N, _D), jnp.float32)] * 4,
    )(x, w1, b1, w2, b2, wq, bq, wk, bk, wv, bv)


def _sc_body(src_hbm, dst_hbm, eb_hbm, q_hbm, k_hbm, v_hbm,
             z128_hbm, numer_hbm, denom_hbm,
             src_v, dst_v, dst2_v, dstn_v, eb_v, a_v, k_v, v_v, w_v,
             acc_n, acc_d, sem_q, sem_k, sem_v, sem_m, sem_s1, sem_s2):
    cid = lax.axis_index("c")
    sid = lax.axis_index("s")
    wid = cid * _NS + sid

    pltpu.sync_copy(z128_hbm, a_v)
    for j in range(_RPT // _ZR):
        pltpu.sync_copy(a_v, acc_n.at[pl.ds(sid * _RPT + j * _ZR, _ZR)])
    pltpu.sync_copy(a_v, acc_d.at[pl.ds(sid * _ZR, _ZR)])
    plsc.subcore_barrier()

    iota = lax.iota(jnp.int32, 16)
    e0 = wid * _EPT

    b0 = pl.multiple_of(e0, 8)
    pltpu.async_copy(src_hbm.at[pl.ds(b0, _B)], src_v, sem_m)
    pltpu.async_copy(dst_hbm.at[pl.ds(b0, _B)], dst_v, sem_m)
    pltpu.async_copy(eb_hbm.at[pl.ds(b0, _B)], eb_v, sem_m)

    def chunk(i, carry):
        base = pl.multiple_of(e0 + i * _B, 8)
        pltpu.make_async_copy(src_hbm.at[pl.ds(base, _B)], src_v, sem_m).wait()
        pltpu.make_async_copy(dst_hbm.at[pl.ds(base, _B)], dst_v, sem_m).wait()
        pltpu.make_async_copy(eb_hbm.at[pl.ds(base, _B)], eb_v, sem_m).wait()
        cq = pltpu.async_copy(q_hbm.at[dst_v], a_v, sem_q)

        @pl.when(i != 0)
        def _():
            pltpu.make_async_copy(v_v, acc_n.at[dstn_v], sem_s1).wait()
            pltpu.make_async_copy(k_v, acc_d.at[dst2_v], sem_s2).wait()

        ck = pltpu.async_copy(k_hbm.at[src_v], k_v, sem_k)
        cv = pltpu.async_copy(v_hbm.at[src_v], v_v, sem_v)
        cq.wait()
        ck.wait()

        def dot_group(g, carry):
            alpha16 = jnp.zeros((16,), jnp.float32)
            for l in range(16):
                e = g * 16 + l
                accs = [a_v[e, pl.ds(c * 16, 16)] * k_v[e, pl.ds(c * 16, 16)]
                        for c in range(4)]
                for c in range(4, 8):
                    accs[c % 4] = accs[c % 4] + (a_v[e, pl.ds(c * 16, 16)] *
                                                 k_v[e, pl.ds(c * 16, 16)])
                s = (accs[0] + accs[1]) + (accs[2] + accs[3])
                for sh in (8, 4, 2, 1):
                    s = s + jnp.take(s, jnp.bitwise_xor(iota, sh))
                alpha16 = jnp.where(iota == l, s, alpha16)
            sl16 = pl.ds(g * 16, 16)
            w_v[sl16] = jnp.exp(alpha16 + eb_v[sl16])
            return carry

        lax.fori_loop(0, _GRP, dot_group, 0)
        cv.wait()

        def scale_group(g, carry):
            sl16 = pl.ds(g * 16, 16)
            w16 = w_v[sl16]
            d16 = dst_v[sl16]
            dstn_v[sl16] = d16
            dst2_v[sl16] = lax.shift_right_logical(d16, 3)
            for l in range(16):
                e = g * 16 + l
                we = w16[l]
                blk_e = jnp.bitwise_and(d16[l], 7)
                for c in range(8):
                    sl = pl.ds(c * 16, 16)
                    v_v[e, sl] = v_v[e, sl] * we
                    wsel = jnp.where(blk_e == c, we, 0.0)
                    k_v[e, sl] = jnp.where(iota == 0, wsel, 0.0)
            return carry

        lax.fori_loop(0, _GRP, scale_group, 0)

        @pl.when(i + 1 < _NCHUNK)
        def _():
            basen = pl.multiple_of(e0 + (i + 1) * _B, 8)
            pltpu.async_copy(src_hbm.at[pl.ds(basen, _B)], src_v, sem_m)
            pltpu.async_copy(dst_hbm.at[pl.ds(basen, _B)], dst_v, sem_m)
            pltpu.async_copy(eb_hbm.at[pl.ds(basen, _B)], eb_v, sem_m)

        pltpu.async_copy(v_v, acc_n.at[dstn_v], sem_s1, add=True)
        pltpu.async_copy(k_v, acc_d.at[dst2_v], sem_s2, add=True)
        return carry

    lax.fori_loop(0, _NCHUNK, chunk, 0)

    pltpu.make_async_copy(v_v, acc_n.at[dstn_v], sem_s1).wait()
    pltpu.make_async_copy(k_v, acc_d.at[dst2_v], sem_s2).wait()

    plsc.subcore_barrier()
    for j in range(_RPT // _ZR):
        r0 = sid * _RPT + j * _ZR
        pltpu.sync_copy(acc_n.at[pl.ds(r0, _ZR)], a_v)
        pltpu.sync_copy(a_v, numer_hbm.at[cid, pl.ds(r0, _ZR)])
    pltpu.sync_copy(acc_d.at[pl.ds(sid * _ZR, _ZR)], k_v)
    pltpu.sync_copy(k_v, denom_hbm.at[cid, pl.ds(sid * _ZR, _ZR)])


_sc_edge = functools.partial(
    pl.kernel,
    out_type=[
        jax.ShapeDtypeStruct((_NC, _NP, _D), jnp.float32),
        jax.ShapeDtypeStruct((_NC, _NP // 8, _D), jnp.float32),
    ],
    mesh=plsc.VectorSubcoreMesh(core_axis_name="c", subcore_axis_name="s"),
    scratch_types=[
        pltpu.VMEM((_B,), jnp.int32),
        pltpu.VMEM((_B,), jnp.int32),
        pltpu.VMEM((_B,), jnp.int32),
        pltpu.VMEM((_B,), jnp.int32),
        pltpu.VMEM((_B,), jnp.float32),
        pltpu.VMEM((_B, _D), jnp.float32),
        pltpu.VMEM((_B, _D), jnp.float32),
        pltpu.VMEM((_B, _D), jnp.float32),
        pltpu.VMEM((_B,), jnp.float32),
        pltpu.VMEM_SHARED((_NP, _D), jnp.float32),
        pltpu.VMEM_SHARED((_NP // 8, _D), jnp.float32),
        pltpu.SemaphoreType.DMA,
        pltpu.SemaphoreType.DMA,
        pltpu.SemaphoreType.DMA,
        pltpu.SemaphoreType.DMA,
        pltpu.SemaphoreType.DMA,
        pltpu.SemaphoreType.DMA,
    ],
)(_sc_body)


def _ebias_body(dist_ref, cnt_ref, se_ref, out_ref):
    dist = dist_ref[...]
    se = jnp.zeros(dist.shape, jnp.float32)
    for t in range(21):
        se = jnp.where(dist == t, se_ref[0, t], se)
    out_ref[...] = se - jnp.log(cnt_ref[...])


def _ebias(dist2d, cnt2d, se_row):
    return pl.pallas_call(
        _ebias_body,
        out_shape=jax.ShapeDtypeStruct((_E // _D, _D), jnp.float32),
    )(dist2d, cnt2d, se_row)


def _combine_ff_body(n0_ref, n1_ref, d0_ref, d1_ref, h8_ref, wsk_ref, bsk_ref,
                     wf1_ref, bf1_ref, wf2_ref, bf2_ref, wo_ref, bo_ref,
                     out_ref):
    den = d0_ref[...] + d1_ref[...] + 1e-16
    o = (n0_ref[...] + n1_ref[...]) / den
    skip = _dot(h8_ref[...], wsk_ref[...]) + bsk_ref[...]
    o = o + skip[1:2, :]
    o = jnp.maximum(_dot(o, wf1_ref[...]) + bf1_ref[...], 0.0)
    o = jnp.maximum(_dot(o, wf2_ref[...]) + bf2_ref[...], 0.0)
    out_ref[...] = _dot(o, wo_ref[...]) + bo_ref[...]


def _combine_ff(n0, n1, d0, d1, h8, wsk, bsk, wf1, bf1, wf2, bf2, wo, bo):
    row = pl.BlockSpec((_BLK, _D), lambda i: (i, 0))
    den = pl.BlockSpec((_BLK, 1), lambda i: (i, 0))
    wsp = pl.BlockSpec((_D, _D), lambda i: (0, 0))
    bsp = pl.BlockSpec((1, _D), lambda i: (0, 0))
    h8sp = pl.BlockSpec((8, _D), lambda i: (0, 0))
    return pl.pallas_call(
        _combine_ff_body,
        grid=(_N // _BLK,),
        in_specs=[row, row, den, den, h8sp,
                  wsp, bsp, wsp, bsp, wsp, bsp, wsp, bsp],
        out_specs=row,
        out_shape=jax.ShapeDtypeStruct((_N, _D), jnp.float32),
    )(n0, n1, d0, d1, h8, wsk, bsk, wf1, bf1, wf2, bf2, wo, bo)


def kernel(x, edge_index, edge_dist, edge_dist_count, batch_idx,
           W_in1, b_in1, W_in2, b_in2, Wq, bq, Wk, bk, Wv, bv,
           W_skip, b_skip, spatial_emb, Wf1, bf1, Wf2, bf2, Wo, bo):
    h, q, k, v = _mlp_qkv(x, W_in1, b_in1.reshape(1, -1), W_in2,
                          b_in2.reshape(1, -1), Wq, bq.reshape(1, -1),
                          Wk, bk.reshape(1, -1), Wv, bv.reshape(1, -1))
    src = edge_index[0]
    dst = edge_index[1]
    se_row = jnp.zeros((1, _D), jnp.float32).at[0, : spatial_emb.shape[0]].set(
        spatial_emb[:, 0])
    ebias = _ebias(edge_dist.reshape(_E // _D, _D),
                   edge_dist_count.reshape(_E // _D, _D), se_row).reshape(_E)
    z128 = jnp.zeros((_ZR, _D), jnp.float32)
    numer, denomp = _sc_edge(src, dst, ebias, q, k, v, z128)
    numer = numer[:, :_N]
    denom = denomp.reshape(_NC, _NP // 8, 8, 16)[:, :, :, 0].reshape(
        _NC, _NP)[:, :_N, None]
    return _combine_ff(numer[0], numer[1], denom[0], denom[1], h[:8],
                       W_skip, b_skip.reshape(1, -1),
                       Wf1, bf1.reshape(1, -1), Wf2, bf2.reshape(1, -1),
                       Wo, bo.reshape(1, -1))


# speedup vs baseline: 11.3581x; 1.2206x over previous
import functools
import math

import jax
import jax.numpy as jnp
from jax import lax
from jax.experimental import pallas as pl
from jax.experimental.pallas import tpu as pltpu
from jax.experimental.pallas import tpu_sc as plsc

_N = 10000
_E = 320000
_D = 128
_NC = 2
_NS = 16
_NW = _NC * _NS
_EPT = _E // _NW
_B = 80
_NCHUNK = _EPT // _B
_GRP = _B // 16
_NP = 10240
_ZR = 80
_RPT = _NP // _NS
_BLK = 1000
_SCALE = 1.0 / math.sqrt(128.0)

_dot = functools.partial(jnp.dot, precision=lax.Precision.HIGHEST,
                         preferred_element_type=jnp.float32)


def _mlp_qkv_body(x_ref, w1_ref, b1_ref, w2_ref, b2_ref, wq_ref, bq_ref,
                  wk_ref, bk_ref, wv_ref, bv_ref,
                  h_ref, q_ref, k_ref, v_ref):
    x = x_ref[...]
    h = jnp.maximum(_dot(x, w1_ref[...]) + b1_ref[...], 0.0)
    h = _dot(h, w2_ref[...]) + b2_ref[...]
    h_ref[...] = h
    q_ref[...] = (_dot(h, wq_ref[...]) + bq_ref[...]) * _SCALE
    k_ref[...] = _dot(h, wk_ref[...]) + bk_ref[...]
    v_ref[...] = _dot(h, wv_ref[...]) + bv_ref[...]


def _mlp_qkv(x, w1, b1, w2, b2, wq, bq, wk, bk, wv, bv):
    row = pl.BlockSpec((_BLK, _D), lambda i: (i, 0))
    wsp = pl.BlockSpec((_D, _D), lambda i: (0, 0))
    bsp = pl.BlockSpec((1, _D), lambda i: (0, 0))
    return pl.pallas_call(
        _mlp_qkv_body,
        grid=(_N // _BLK,),
        in_specs=[row, wsp, bsp, wsp, bsp, wsp, bsp, wsp, bsp, wsp, bsp],
        out_specs=[row, row, row, row],
        out_shape=[jax.ShapeDtypeStruct((_N, _D), jnp.float32)] * 4,
    )(x, w1, b1, w2, b2, wq, bq, wk, bk, wv, bv)


def _sc_body(src_hbm, dst_hbm, eb_hbm, q_hbm, k_hbm, v_hbm,
             z128_hbm, numer_hbm, denom_hbm,
             src_v, dst_v, dst2_v, dstn_v, eb_v, a_v, k_v, v_v, w_v,
             acc_n, acc_d, sem_q, sem_k, sem_v, sem_m, sem_s1, sem_s2):
    cid = lax.axis_index("c")
    sid = lax.axis_index("s")
    wid = cid * _NS + sid

    pltpu.sync_copy(z128_hbm, a_v)
    for j in range(_RPT // _ZR):
        pltpu.sync_copy(a_v, acc_n.at[pl.ds(sid * _RPT + j * _ZR, _ZR)])
    pltpu.sync_copy(a_v, acc_d.at[pl.ds(sid * _ZR, _ZR)])
    plsc.subcore_barrier()

    iota = lax.iota(jnp.int32, 16)
    e0 = wid * _EPT

    b0 = pl.multiple_of(e0, 8)
    pltpu.async_copy(src_hbm.at[pl.ds(b0, _B)], src_v, sem_m)
    pltpu.async_copy(dst_hbm.at[pl.ds(b0, _B)], dst_v, sem_m)
    pltpu.async_copy(eb_hbm.at[pl.ds(b0, _B)], eb_v, sem_m)

    def chunk(i, carry):
        base = pl.multiple_of(e0 + i * _B, 8)
        pltpu.make_async_copy(src_hbm.at[pl.ds(base, _B)], src_v, sem_m).wait()
        pltpu.make_async_copy(dst_hbm.at[pl.ds(base, _B)], dst_v, sem_m).wait()
        pltpu.make_async_copy(eb_hbm.at[pl.ds(base, _B)], eb_v, sem_m).wait()
        cq = pltpu.async_copy(q_hbm.at[dst_v], a_v, sem_q)

        @pl.when(i != 0)
        def _():
            pltpu.make_async_copy(v_v, acc_n.at[dstn_v], sem_s1).wait()
            pltpu.make_async_copy(k_v, acc_d.at[dst2_v], sem_s2).wait()

        ck = pltpu.async_copy(k_hbm.at[src_v], k_v, sem_k)
        cv = pltpu.async_copy(v_hbm.at[src_v], v_v, sem_v)
        cq.wait()
        ck.wait()

        def dot_group(g, carry):
            alpha16 = jnp.zeros((16,), jnp.float32)
            for l in range(16):
                e = g * 16 + l
                accs = [a_v[e, pl.ds(c * 16, 16)] * k_v[e, pl.ds(c * 16, 16)]
                        for c in range(4)]
                for c in range(4, 8):
                    accs[c % 4] = accs[c % 4] + (a_v[e, pl.ds(c * 16, 16)] *
                                                 k_v[e, pl.ds(c * 16, 16)])
                s = (accs[0] + accs[1]) + (accs[2] + accs[3])
                for sh in (8, 4, 2, 1):
                    s = s + jnp.take(s, jnp.bitwise_xor(iota, sh))
                alpha16 = jnp.where(iota == l, s, alpha16)
            sl16 = pl.ds(g * 16, 16)
            w_v[sl16] = jnp.exp(alpha16 + eb_v[sl16])
            d16 = dst_v[sl16]
            dstn_v[sl16] = d16
            dst2_v[sl16] = lax.shift_right_logical(d16, 3)
            return carry

        lax.fori_loop(0, _GRP, dot_group, 0)

        @pl.when(i + 1 < _NCHUNK)
        def _():
            basen = pl.multiple_of(e0 + (i + 1) * _B, 8)
            pltpu.async_copy(src_hbm.at[pl.ds(basen, _B)], src_v, sem_m)
            pltpu.async_copy(dst_hbm.at[pl.ds(basen, _B)], dst_v, sem_m)
            pltpu.async_copy(eb_hbm.at[pl.ds(basen, _B)], eb_v, sem_m)

        cv.wait()

        def scale_group(g, carry):
            sl16 = pl.ds(g * 16, 16)
            w16 = w_v[sl16]
            d16 = dstn_v[sl16]
            for l in range(16):
                e = g * 16 + l
                we = w16[l]
                blk_e = jnp.bitwise_and(d16[l], 7)
                for c in range(8):
                    sl = pl.ds(c * 16, 16)
                    v_v[e, sl] = v_v[e, sl] * we
                    wsel = jnp.where(blk_e == c, we, 0.0)
                    k_v[e, sl] = jnp.where(iota == 0, wsel, 0.0)
            return carry

        lax.fori_loop(0, _GRP, scale_group, 0)
        pltpu.async_copy(v_v, acc_n.at[dstn_v], sem_s1, add=True)
        pltpu.async_copy(k_v, acc_d.at[dst2_v], sem_s2, add=True)
        return carry

    lax.fori_loop(0, _NCHUNK, chunk, 0)

    pltpu.make_async_copy(v_v, acc_n.at[dstn_v], sem_s1).wait()
    pltpu.make_async_copy(k_v, acc_d.at[dst2_v], sem_s2).wait()

    plsc.subcore_barrier()
    for j in range(_RPT // _ZR):
        r0 = sid * _RPT + j * _ZR
        pltpu.sync_copy(acc_n.at[pl.ds(r0, _ZR)], a_v)
        pltpu.sync_copy(a_v, numer_hbm.at[cid, pl.ds(r0, _ZR)])
    pltpu.sync_copy(acc_d.at[pl.ds(sid * _ZR, _ZR)], k_v)
    pltpu.sync_copy(k_v, denom_hbm.at[cid, pl.ds(sid * _ZR, _ZR)])


_sc_edge = functools.partial(
    pl.kernel,
    out_type=[
        jax.ShapeDtypeStruct((_NC, _NP, _D), jnp.float32),
        jax.ShapeDtypeStruct((_NC, _NP // 8, _D), jnp.float32),
    ],
    mesh=plsc.VectorSubcoreMesh(core_axis_name="c", subcore_axis_name="s"),
    scratch_types=[
        pltpu.VMEM((_B,), jnp.int32),
        pltpu.VMEM((_B,), jnp.int32),
        pltpu.VMEM((_B,), jnp.int32),
        pltpu.VMEM((_B,), jnp.int32),
        pltpu.VMEM((_B,), jnp.float32),
        pltpu.VMEM((_B, _D), jnp.float32),
        pltpu.VMEM((_B, _D), jnp.float32),
        pltpu.VMEM((_B, _D), jnp.float32),
        pltpu.VMEM((_B,), jnp.float32),
        pltpu.VMEM_SHARED((_NP, _D), jnp.float32),
        pltpu.VMEM_SHARED((_NP // 8, _D), jnp.float32),
        pltpu.SemaphoreType.DMA,
        pltpu.SemaphoreType.DMA,
        pltpu.SemaphoreType.DMA,
        pltpu.SemaphoreType.DMA,
        pltpu.SemaphoreType.DMA,
        pltpu.SemaphoreType.DMA,
    ],
)(_sc_body)


def _ebias_body(dist_ref, cnt_ref, se_ref, out_ref):
    dist = dist_ref[...]
    se = jnp.zeros(dist.shape, jnp.float32)
    for t in range(21):
        se = jnp.where(dist == t, se_ref[0, t], se)
    out_ref[...] = se - jnp.log(cnt_ref[...])


def _ebias(dist2d, cnt2d, se_row):
    return pl.pallas_call(
        _ebias_body,
        out_shape=jax.ShapeDtypeStruct((_E // _D, _D), jnp.float32),
    )(dist2d, cnt2d, se_row)


def _combine_ff_body(n0_ref, n1_ref, d0_ref, d1_ref, h8_ref, wsk_ref, bsk_ref,
                     wf1_ref, bf1_ref, wf2_ref, bf2_ref, wo_ref, bo_ref,
                     out_ref):
    den = d0_ref[...] + d1_ref[...] + 1e-16
    o = (n0_ref[...] + n1_ref[...]) / den
    skip = _dot(h8_ref[...], wsk_ref[...]) + bsk_ref[...]
    o = o + skip[1:2, :]
    o = jnp.maximum(_dot(o, wf1_ref[...]) + bf1_ref[...], 0.0)
    o = jnp.maximum(_dot(o, wf2_ref[...]) + bf2_ref[...], 0.0)
    out_ref[...] = _dot(o, wo_ref[...]) + bo_ref[...]


def _combine_ff(n0, n1, d0, d1, h8, wsk, bsk, wf1, bf1, wf2, bf2, wo, bo):
    row = pl.BlockSpec((_BLK, _D), lambda i: (i, 0))
    den = pl.BlockSpec((_BLK, 1), lambda i: (i, 0))
    wsp = pl.BlockSpec((_D, _D), lambda i: (0, 0))
    bsp = pl.BlockSpec((1, _D), lambda i: (0, 0))
    h8sp = pl.BlockSpec((8, _D), lambda i: (0, 0))
    return pl.pallas_call(
        _combine_ff_body,
        grid=(_N // _BLK,),
        in_specs=[row, row, den, den, h8sp,
                  wsp, bsp, wsp, bsp, wsp, bsp, wsp, bsp],
        out_specs=row,
        out_shape=jax.ShapeDtypeStruct((_N, _D), jnp.float32),
    )(n0, n1, d0, d1, h8, wsk, bsk, wf1, bf1, wf2, bf2, wo, bo)


def kernel(x, edge_index, edge_dist, edge_dist_count, batch_idx,
           W_in1, b_in1, W_in2, b_in2, Wq, bq, Wk, bk, Wv, bv,
           W_skip, b_skip, spatial_emb, Wf1, bf1, Wf2, bf2, Wo, bo):
    h, q, k, v = _mlp_qkv(x, W_in1, b_in1.reshape(1, -1), W_in2,
                          b_in2.reshape(1, -1), Wq, bq.reshape(1, -1),
                          Wk, bk.reshape(1, -1), Wv, bv.reshape(1, -1))
    src = edge_index[0]
    dst = edge_index[1]
    se_row = jnp.zeros((1, _D), jnp.float32).at[0, : spatial_emb.shape[0]].set(
        spatial_emb[:, 0])
    ebias = _ebias(edge_dist.reshape(_E // _D, _D),
                   edge_dist_count.reshape(_E // _D, _D), se_row).reshape(_E)
    z128 = jnp.zeros((_ZR, _D), jnp.float32)
    numer, denomp = _sc_edge(src, dst, ebias, q, k, v, z128)
    numer = numer[:, :_N]
    denom = denomp.reshape(_NC, _NP // 8, 8, 16)[:, :, :, 0].reshape(
        _NC, _NP)[:, :_N, None]
    return _combine_ff(numer[0], numer[1], denom[0], denom[1], h[:8],
                       W_skip, b_skip.reshape(1, -1),
                       Wf1, bf1.reshape(1, -1), Wf2, bf2.reshape(1, -1),
                       Wo, bo.reshape(1, -1))
